# baseline (speedup 1.0000x reference)
import functools

import jax
import jax.numpy as jnp
from jax.experimental import pallas as pl

_BM = 256


def _proj_body(x_ref, w_ref, wh_ref):
    wh_ref[0] = jnp.dot(x_ref[...], w_ref[0], preferred_element_type=jnp.float32)


def _attn_body(adj_ref, wh_ref, asrc_ref, adst_ref, out_ref, *, bm, heads, hid):
    i = pl.program_id(0)
    adj = adj_ref[...]
    for h in range(heads):
        wh = wh_ref[h]
        asrc = asrc_ref[h][:, None]
        adst = adst_ref[h][:, None]
        rows = wh_ref[h, pl.ds(i * bm, bm), :]
        f_src = jnp.dot(rows, asrc, preferred_element_type=jnp.float32)
        f_dst = jnp.dot(wh, adst, preferred_element_type=jnp.float32)
        f_dst = f_dst.reshape(1, -1)
        mh = jnp.max(f_dst)
        t = f_src + mh
        c = jnp.maximum(t, 0.2 * t)
        a_row = jnp.exp(f_src - c)
        c_row = jnp.exp(0.2 * f_src - c)
        b_col = jnp.exp(f_dst)
        d_col = jnp.exp(0.2 * f_dst)
        p = adj * jnp.maximum(a_row * b_col, c_row * d_col)
        denom = jnp.sum(p, axis=1, keepdims=True)
        pv = jnp.dot(p, wh, preferred_element_type=jnp.float32)
        mean_wh = jnp.mean(wh, axis=0, keepdims=True)
        o = jnp.where(denom > 0.0, pv / denom, mean_wh)
        o = jnp.where(o > 0.0, o, jnp.exp(o) - 1.0)
        out_ref[:, h * hid : (h + 1) * hid] = o


def _gat_layer(x, adj, W, a_src, a_dst):
    n = x.shape[0]
    fin = x.shape[1]
    heads, _, hid = W.shape
    bm = min(_BM, n)

    wh = pl.pallas_call(
        _proj_body,
        grid=(heads,),
        in_specs=[
            pl.BlockSpec((n, fin), lambda h: (0, 0)),
            pl.BlockSpec((1, fin, hid), lambda h: (h, 0, 0)),
        ],
        out_specs=pl.BlockSpec((1, n, hid), lambda h: (h, 0, 0)),
        out_shape=jax.ShapeDtypeStruct((heads, n, hid), jnp.float32),
    )(x, W)

    out = pl.pallas_call(
        functools.partial(_attn_body, bm=bm, heads=heads, hid=hid),
        grid=(n // bm,),
        in_specs=[
            pl.BlockSpec((bm, n), lambda i: (i, 0)),
            pl.BlockSpec((heads, n, hid), lambda i: (0, 0, 0)),
            pl.BlockSpec((heads, hid), lambda i: (0, 0)),
            pl.BlockSpec((heads, hid), lambda i: (0, 0)),
        ],
        out_specs=pl.BlockSpec((bm, heads * hid), lambda i: (i, 0)),
        out_shape=jax.ShapeDtypeStruct((n, heads * hid), jnp.float32),
    )(adj, wh, a_src, a_dst)
    return out


@jax.jit
def kernel(features, adj, W1, a_src1, a_dst1, W2, a_src2, a_dst2, W3, a_src3, a_dst3):
    x = _gat_layer(features, adj, W1, a_src1, a_dst1)
    x = _gat_layer(x, adj, W2, a_src2, a_dst2)
    x = _gat_layer(x, adj, W3, a_src3, a_dst3)
    return x


# speedup vs baseline: 1.1927x; 1.1927x over previous
import functools

import jax
import jax.numpy as jnp
from jax.experimental import pallas as pl

_BM = 256


def _proj_body(x_ref, w_ref, whe_ref, mean_ref, *, hid):
    n = x_ref.shape[0]
    wh = jnp.dot(x_ref[...], w_ref[0], preferred_element_type=jnp.float32)
    whe_ref[0, :, :hid] = wh
    col = jax.lax.broadcasted_iota(jnp.int32, (n, hid), 1)
    whe_ref[0, :, hid:] = jnp.where(col == 0, jnp.float32(1.0), jnp.float32(0.0))
    mean_ref[0] = jnp.mean(wh, axis=0, keepdims=True)


def _attn_body(adj_ref, whe_ref, asrc_ref, adst_ref, mean_ref, out_ref, *, bm, heads, hid):
    i = pl.program_id(0)
    adj = adj_ref[...]
    for h in range(heads):
        rows = whe_ref[h, pl.ds(i * bm, bm), :hid]
        wh = whe_ref[h, :, :hid]
        asrc = asrc_ref[h][:, None]
        adst = adst_ref[h][:, None]
        f_src = jnp.dot(rows, asrc, preferred_element_type=jnp.float32)
        f_dst = jnp.dot(wh, adst, preferred_element_type=jnp.float32)
        f_dst = f_dst.reshape(1, -1)
        mh = jnp.max(f_dst)
        t = f_src + mh
        c = jnp.maximum(t, 0.2 * t)
        a_row = jnp.exp(f_src - c)
        c_row = jnp.exp(0.2 * f_src - c)
        b_col = jnp.exp(f_dst)
        d_col = jnp.exp(0.2 * f_dst)
        ab = jnp.dot(a_row, b_col, preferred_element_type=jnp.float32)
        cd = jnp.dot(c_row, d_col, preferred_element_type=jnp.float32)
        p = adj * jnp.maximum(ab, cd)
        pv = jnp.dot(p, whe_ref[h], preferred_element_type=jnp.float32)
        denom = pv[:, hid : hid + 1]
        o = jnp.where(denom > 0.0, pv[:, :hid] / denom, mean_ref[h])
        o = jnp.where(o > 0.0, o, jnp.exp(o) - 1.0)
        out_ref[:, h * hid : (h + 1) * hid] = o


def _gat_layer(x, adj, W, a_src, a_dst):
    n = x.shape[0]
    fin = x.shape[1]
    heads, _, hid = W.shape
    bm = min(_BM, n)

    whe, wh_mean = pl.pallas_call(
        functools.partial(_proj_body, hid=hid),
        grid=(heads,),
        in_specs=[
            pl.BlockSpec((n, fin), lambda h: (0, 0)),
            pl.BlockSpec((1, fin, hid), lambda h: (h, 0, 0)),
        ],
        out_specs=[
            pl.BlockSpec((1, n, 2 * hid), lambda h: (h, 0, 0)),
            pl.BlockSpec((1, 1, hid), lambda h: (h, 0, 0)),
        ],
        out_shape=[
            jax.ShapeDtypeStruct((heads, n, 2 * hid), jnp.float32),
            jax.ShapeDtypeStruct((heads, 1, hid), jnp.float32),
        ],
    )(x, W)

    out = pl.pallas_call(
        functools.partial(_attn_body, bm=bm, heads=heads, hid=hid),
        grid=(n // bm,),
        in_specs=[
            pl.BlockSpec((bm, n), lambda i: (i, 0)),
            pl.BlockSpec((heads, n, 2 * hid), lambda i: (0, 0, 0)),
            pl.BlockSpec((heads, hid), lambda i: (0, 0)),
            pl.BlockSpec((heads, hid), lambda i: (0, 0)),
            pl.BlockSpec((heads, 1, hid), lambda i: (0, 0, 0)),
        ],
        out_specs=pl.BlockSpec((bm, heads * hid), lambda i: (i, 0)),
        out_shape=jax.ShapeDtypeStruct((n, heads * hid), jnp.float32),
    )(adj, whe, a_src, a_dst, wh_mean)
    return out


@jax.jit
def kernel(features, adj, W1, a_src1, a_dst1, W2, a_src2, a_dst2, W3, a_src3, a_dst3):
    x = _gat_layer(features, adj, W1, a_src1, a_dst1)
    x = _gat_layer(x, adj, W2, a_src2, a_dst2)
    x = _gat_layer(x, adj, W3, a_src3, a_dst3)
    return x


# speedup vs baseline: 1.3224x; 1.1088x over previous
import functools

import jax
import jax.numpy as jnp
from jax.experimental import pallas as pl

_BM = 256


def _proj_body(x_ref, w_ref, wh_ref, whe_ref, mean_ref, *, hid):
    n = x_ref.shape[0]
    wh = jnp.dot(x_ref[...], w_ref[0], preferred_element_type=jnp.float32)
    wh_ref[0] = wh
    whe_ref[0, :, :hid] = wh.astype(jnp.bfloat16)
    col = jax.lax.broadcasted_iota(jnp.int32, (n, hid), 1)
    whe_ref[0, :, hid:] = jnp.where(col == 0, 1.0, 0.0).astype(jnp.bfloat16)
    mean_ref[0] = jnp.mean(wh, axis=0, keepdims=True)


def _attn_body(adj_ref, wh_ref, whe_ref, asrc_ref, adst_ref, mean_ref, out_ref,
               *, bm, heads, hid):
    i = pl.program_id(0)
    adj = adj_ref[...]
    for h in range(heads):
        rows = wh_ref[h, pl.ds(i * bm, bm), :]
        wh = wh_ref[h]
        asrc = asrc_ref[h][:, None]
        adst = adst_ref[h][:, None]
        f_src = jnp.dot(rows, asrc, preferred_element_type=jnp.float32)
        f_dst = jnp.dot(wh, adst, preferred_element_type=jnp.float32)
        f_dst = f_dst.reshape(1, -1)
        mh = jnp.max(f_dst)
        t = f_src + mh
        c = jnp.maximum(t, 0.2 * t)
        u = f_src - c
        v = 0.2 * f_src - c
        fd2 = 0.2 * f_dst
        e = jnp.maximum(u + f_dst, v + fd2)
        p = (adj * jnp.exp(e)).astype(jnp.bfloat16)
        pv = jnp.dot(p, whe_ref[h], preferred_element_type=jnp.float32)
        denom = pv[:, hid : hid + 1]
        o = jnp.where(denom > 0.0, pv[:, :hid] / denom, mean_ref[h])
        o = jnp.where(o > 0.0, o, jnp.exp(o) - 1.0)
        out_ref[:, h * hid : (h + 1) * hid] = o


def _gat_layer(x, adj, W, a_src, a_dst):
    n = x.shape[0]
    fin = x.shape[1]
    heads, _, hid = W.shape
    bm = min(_BM, n)

    wh, whe, wh_mean = pl.pallas_call(
        functools.partial(_proj_body, hid=hid),
        grid=(heads,),
        in_specs=[
            pl.BlockSpec((n, fin), lambda h: (0, 0)),
            pl.BlockSpec((1, fin, hid), lambda h: (h, 0, 0)),
        ],
        out_specs=[
            pl.BlockSpec((1, n, hid), lambda h: (h, 0, 0)),
            pl.BlockSpec((1, n, 2 * hid), lambda h: (h, 0, 0)),
            pl.BlockSpec((1, 1, hid), lambda h: (h, 0, 0)),
        ],
        out_shape=[
            jax.ShapeDtypeStruct((heads, n, hid), jnp.float32),
            jax.ShapeDtypeStruct((heads, n, 2 * hid), jnp.bfloat16),
            jax.ShapeDtypeStruct((heads, 1, hid), jnp.float32),
        ],
    )(x, W)

    out = pl.pallas_call(
        functools.partial(_attn_body, bm=bm, heads=heads, hid=hid),
        grid=(n // bm,),
        in_specs=[
            pl.BlockSpec((bm, n), lambda i: (i, 0)),
            pl.BlockSpec((heads, n, hid), lambda i: (0, 0, 0)),
            pl.BlockSpec((heads, n, 2 * hid), lambda i: (0, 0, 0)),
            pl.BlockSpec((heads, hid), lambda i: (0, 0)),
            pl.BlockSpec((heads, hid), lambda i: (0, 0)),
            pl.BlockSpec((heads, 1, hid), lambda i: (0, 0, 0)),
        ],
        out_specs=pl.BlockSpec((bm, heads * hid), lambda i: (i, 0)),
        out_shape=jax.ShapeDtypeStruct((n, heads * hid), jnp.float32),
    )(adj, wh, whe, a_src, a_dst, wh_mean)
    return out


@jax.jit
def kernel(features, adj, W1, a_src1, a_dst1, W2, a_src2, a_dst2, W3, a_src3, a_dst3):
    x = _gat_layer(features, adj, W1, a_src1, a_dst1)
    x = _gat_layer(x, adj, W2, a_src2, a_dst2)
    x = _gat_layer(x, adj, W3, a_src3, a_dst3)
    return x


# speedup vs baseline: 2.0270x; 1.5328x over previous
import functools

import jax
import jax.numpy as jnp
from jax.experimental import pallas as pl

_BM = 256


def _proj_body(x_ref, w_ref, asrc_ref, adst_ref,
               whe_ref, u_ref, v_ref, fr_ref, f2_ref, mean_ref, *, hid):
    n = x_ref.shape[0]
    h = pl.program_id(0)
    wh = jnp.dot(x_ref[...], w_ref[0], preferred_element_type=jnp.float32)
    whe_ref[0, :, :hid] = wh.astype(jnp.bfloat16)
    col = jax.lax.broadcasted_iota(jnp.int32, (n, hid), 1)
    whe_ref[0, :, hid:] = jnp.where(col == 0, 1.0, 0.0).astype(jnp.bfloat16)
    asrc = asrc_ref[h][:, None]
    adst = adst_ref[h][:, None]
    f_src = jnp.dot(wh, asrc, preferred_element_type=jnp.float32)
    f_dst = jnp.dot(wh, adst, preferred_element_type=jnp.float32)
    mh = jnp.max(f_dst)
    t = f_src + mh
    c = jnp.maximum(t, 0.2 * t)
    u_ref[0] = f_src - c
    v_ref[0] = 0.2 * f_src - c
    fr = f_dst.reshape(1, -1)
    fr_ref[0] = fr
    f2_ref[0] = 0.2 * fr
    mean_ref[0] = jnp.mean(wh, axis=0, keepdims=True)


def _attn_body(adj_ref, whe_ref, u_ref, v_ref, fr_ref, f2_ref, mean_ref,
               out_ref, *, bm, heads, hid):
    i = pl.program_id(0)
    adj = adj_ref[...]
    for h in range(heads):
        u = u_ref[h, pl.ds(i * bm, bm), :]
        v = v_ref[h, pl.ds(i * bm, bm), :]
        fr = fr_ref[h]
        f2 = f2_ref[h]
        e = jnp.maximum(u + fr, v + f2)
        p = (adj * jnp.exp(e)).astype(jnp.bfloat16)
        pv = jnp.dot(p, whe_ref[h], preferred_element_type=jnp.float32)
        denom = pv[:, hid : hid + 1]
        o = jnp.where(denom > 0.0, pv[:, :hid] / denom, mean_ref[h])
        o = jnp.where(o > 0.0, o, jnp.exp(o) - 1.0)
        out_ref[:, h * hid : (h + 1) * hid] = o


def _gat_layer(x, adj, W, a_src, a_dst):
    n = x.shape[0]
    fin = x.shape[1]
    heads, _, hid = W.shape
    bm = min(_BM, n)

    whe, u, v, fr, f2, wh_mean = pl.pallas_call(
        functools.partial(_proj_body, hid=hid),
        grid=(heads,),
        in_specs=[
            pl.BlockSpec((n, fin), lambda h: (0, 0)),
            pl.BlockSpec((1, fin, hid), lambda h: (h, 0, 0)),
            pl.BlockSpec((heads, hid), lambda h: (0, 0)),
            pl.BlockSpec((heads, hid), lambda h: (0, 0)),
        ],
        out_specs=[
            pl.BlockSpec((1, n, 2 * hid), lambda h: (h, 0, 0)),
            pl.BlockSpec((1, n, 1), lambda h: (h, 0, 0)),
            pl.BlockSpec((1, n, 1), lambda h: (h, 0, 0)),
            pl.BlockSpec((1, 1, n), lambda h: (h, 0, 0)),
            pl.BlockSpec((1, 1, n), lambda h: (h, 0, 0)),
            pl.BlockSpec((1, 1, hid), lambda h: (h, 0, 0)),
        ],
        out_shape=[
            jax.ShapeDtypeStruct((heads, n, 2 * hid), jnp.bfloat16),
            jax.ShapeDtypeStruct((heads, n, 1), jnp.float32),
            jax.ShapeDtypeStruct((heads, n, 1), jnp.float32),
            jax.ShapeDtypeStruct((heads, 1, n), jnp.float32),
            jax.ShapeDtypeStruct((heads, 1, n), jnp.float32),
            jax.ShapeDtypeStruct((heads, 1, hid), jnp.float32),
        ],
    )(x, W, a_src, a_dst)

    out = pl.pallas_call(
        functools.partial(_attn_body, bm=bm, heads=heads, hid=hid),
        grid=(n // bm,),
        in_specs=[
            pl.BlockSpec((bm, n), lambda i: (i, 0)),
            pl.BlockSpec((heads, n, 2 * hid), lambda i: (0, 0, 0)),
            pl.BlockSpec((heads, n, 1), lambda i: (0, 0, 0)),
            pl.BlockSpec((heads, n, 1), lambda i: (0, 0, 0)),
            pl.BlockSpec((heads, 1, n), lambda i: (0, 0, 0)),
            pl.BlockSpec((heads, 1, n), lambda i: (0, 0, 0)),
            pl.BlockSpec((heads, 1, hid), lambda i: (0, 0, 0)),
        ],
        out_specs=pl.BlockSpec((bm, heads * hid), lambda i: (i, 0)),
        out_shape=jax.ShapeDtypeStruct((n, heads * hid), jnp.float32),
    )(adj, whe, u, v, fr, f2, wh_mean)
    return out


@jax.jit
def kernel(features, adj, W1, a_src1, a_dst1, W2, a_src2, a_dst2, W3, a_src3, a_dst3):
    x = _gat_layer(features, adj, W1, a_src1, a_dst1)
    x = _gat_layer(x, adj, W2, a_src2, a_dst2)
    x = _gat_layer(x, adj, W3, a_src3, a_dst3)
    return x


# speedup vs baseline: 2.2101x; 1.0904x over previous
import functools

import jax
import jax.numpy as jnp
from jax.experimental import pallas as pl

_BM = 256


def _proj_body(x_ref, w_ref, asrc_ref, adst_ref,
               whe_ref, ea_ref, ec_ref, eb_ref, ed_ref, mean_ref, *, hid):
    n = x_ref.shape[0]
    h = pl.program_id(0)
    wh = jnp.dot(x_ref[...], w_ref[0], preferred_element_type=jnp.float32)
    whe_ref[0, :, :hid] = wh.astype(jnp.bfloat16)
    col = jax.lax.broadcasted_iota(jnp.int32, (n, hid), 1)
    whe_ref[0, :, hid:] = jnp.where(col == 0, 1.0, 0.0).astype(jnp.bfloat16)
    asrc = asrc_ref[h][:, None]
    adst = adst_ref[h][:, None]
    f_src = jnp.dot(wh, asrc, preferred_element_type=jnp.float32)
    f_dst = jnp.dot(wh, adst, preferred_element_type=jnp.float32)
    mh = jnp.max(f_dst)
    t = f_src + mh
    c = jnp.maximum(t, 0.2 * t)
    ea_ref[0] = jnp.exp(f_src - c)
    ec_ref[0] = jnp.exp(0.2 * f_src - c)
    fr = f_dst.reshape(1, -1)
    eb_ref[0] = jnp.exp(fr)
    ed_ref[0] = jnp.exp(0.2 * fr)
    mean_ref[0] = jnp.mean(wh, axis=0, keepdims=True)


def _attn_body(adj_ref, whe_ref, ea_ref, ec_ref, eb_ref, ed_ref, mean_ref,
               out_ref, *, bm, heads, hid):
    i = pl.program_id(0)
    adj = adj_ref[...]
    for h in range(heads):
        ea = ea_ref[h, pl.ds(i * bm, bm), :]
        ec = ec_ref[h, pl.ds(i * bm, bm), :]
        eb = eb_ref[h]
        ed = ed_ref[h]
        q = jnp.maximum(ea * eb, ec * ed)
        p = adj * q.astype(jnp.bfloat16)
        pv = jnp.dot(p, whe_ref[h], preferred_element_type=jnp.float32)
        denom = pv[:, hid : hid + 1]
        o = jnp.where(denom > 0.0, pv[:, :hid] / denom, mean_ref[h])
        o = jnp.where(o > 0.0, o, jnp.exp(o) - 1.0)
        out_ref[:, h * hid : (h + 1) * hid] = o


def _gat_layer(x, adj_bf, W, a_src, a_dst):
    n = x.shape[0]
    fin = x.shape[1]
    heads, _, hid = W.shape
    bm = min(_BM, n)

    whe, ea, ec, eb, ed, wh_mean = pl.pallas_call(
        functools.partial(_proj_body, hid=hid),
        grid=(heads,),
        in_specs=[
            pl.BlockSpec((n, fin), lambda h: (0, 0)),
            pl.BlockSpec((1, fin, hid), lambda h: (h, 0, 0)),
            pl.BlockSpec((heads, hid), lambda h: (0, 0)),
            pl.BlockSpec((heads, hid), lambda h: (0, 0)),
        ],
        out_specs=[
            pl.BlockSpec((1, n, 2 * hid), lambda h: (h, 0, 0)),
            pl.BlockSpec((1, n, 1), lambda h: (h, 0, 0)),
            pl.BlockSpec((1, n, 1), lambda h: (h, 0, 0)),
            pl.BlockSpec((1, 1, n), lambda h: (h, 0, 0)),
            pl.BlockSpec((1, 1, n), lambda h: (h, 0, 0)),
            pl.BlockSpec((1, 1, hid), lambda h: (h, 0, 0)),
        ],
        out_shape=[
            jax.ShapeDtypeStruct((heads, n, 2 * hid), jnp.bfloat16),
            jax.ShapeDtypeStruct((heads, n, 1), jnp.float32),
            jax.ShapeDtypeStruct((heads, n, 1), jnp.float32),
            jax.ShapeDtypeStruct((heads, 1, n), jnp.float32),
            jax.ShapeDtypeStruct((heads, 1, n), jnp.float32),
            jax.ShapeDtypeStruct((heads, 1, hid), jnp.float32),
        ],
    )(x, W, a_src, a_dst)

    out = pl.pallas_call(
        functools.partial(_attn_body, bm=bm, heads=heads, hid=hid),
        grid=(n // bm,),
        in_specs=[
            pl.BlockSpec((bm, n), lambda i: (i, 0)),
            pl.BlockSpec((heads, n, 2 * hid), lambda i: (0, 0, 0)),
            pl.BlockSpec((heads, n, 1), lambda i: (0, 0, 0)),
            pl.BlockSpec((heads, n, 1), lambda i: (0, 0, 0)),
            pl.BlockSpec((heads, 1, n), lambda i: (0, 0, 0)),
            pl.BlockSpec((heads, 1, n), lambda i: (0, 0, 0)),
            pl.BlockSpec((heads, 1, hid), lambda i: (0, 0, 0)),
        ],
        out_specs=pl.BlockSpec((bm, heads * hid), lambda i: (i, 0)),
        out_shape=jax.ShapeDtypeStruct((n, heads * hid), jnp.float32),
    )(adj_bf, whe, ea, ec, eb, ed, wh_mean)
    return out


@jax.jit
def kernel(features, adj, W1, a_src1, a_dst1, W2, a_src2, a_dst2, W3, a_src3, a_dst3):
    adj_bf = adj.astype(jnp.bfloat16)
    x = _gat_layer(features, adj_bf, W1, a_src1, a_dst1)
    x = _gat_layer(x, adj_bf, W2, a_src2, a_dst2)
    x = _gat_layer(x, adj_bf, W3, a_src3, a_dst3)
    return x


# speedup vs baseline: 2.7833x; 1.2593x over previous
import functools

import jax
import jax.numpy as jnp
from jax.experimental import pallas as pl

_BM = 256


def _proj_body(x_ref, w_ref, asrc_ref, adst_ref,
               whe_ref, ea_ref, ec_ref, eb_ref, ed_ref, mean_ref, *, hid):
    n = x_ref.shape[0]
    h = pl.program_id(0)
    wh = jnp.dot(x_ref[...], w_ref[0], preferred_element_type=jnp.float32)
    whe_ref[0, :, :hid] = wh.astype(jnp.bfloat16)
    col = jax.lax.broadcasted_iota(jnp.int32, (n, hid), 1)
    whe_ref[0, :, hid:] = jnp.where(col == 0, 1.0, 0.0).astype(jnp.bfloat16)
    asrc = asrc_ref[h][:, None]
    adst = adst_ref[h][:, None]
    aboth = jnp.concatenate([asrc, adst], axis=1)
    fs = jnp.dot(wh, aboth, preferred_element_type=jnp.float32)
    f_src = fs[:, 0:1]
    f_dst = fs[:, 1:2]
    mh = jnp.max(f_dst)
    t = f_src + mh
    c = jnp.maximum(t, 0.2 * t)
    ea_ref[0] = jnp.exp(f_src - c).astype(jnp.bfloat16)
    ec_ref[0] = jnp.exp(0.2 * f_src - c).astype(jnp.bfloat16)
    eb_ref[0] = jnp.exp(f_dst).astype(jnp.bfloat16)
    ed_ref[0] = jnp.exp(0.2 * f_dst).astype(jnp.bfloat16)
    mean_ref[0] = jnp.mean(wh, axis=0, keepdims=True)


def _attn_body(adj_ref, whe_ref, ea_ref, ec_ref, eb_ref, ed_ref, mean_ref,
               out_ref, *, bm, heads, hid):
    i = pl.program_id(0)
    adj = adj_ref[...]
    for h in range(heads):
        ea = ea_ref[h, pl.ds(i * bm, bm), :]
        ec = ec_ref[h, pl.ds(i * bm, bm), :]
        eb = eb_ref[h]
        ed = ed_ref[h]
        q = jnp.maximum(ea * eb, ec * ed)
        p = adj * q
        pv = jnp.dot(p, whe_ref[h], preferred_element_type=jnp.float32)
        denom = pv[:, hid : hid + 1]
        o = jnp.where(denom > 0.0, pv[:, :hid] / denom, mean_ref[h])
        o = jnp.where(o > 0.0, o, jnp.exp(o) - 1.0)
        out_ref[:, h * hid : (h + 1) * hid] = o


def _gat_layer(x, adj_bf, W, a_src, a_dst):
    n = x.shape[0]
    fin = x.shape[1]
    heads, _, hid = W.shape
    bm = min(_BM, n)

    whe, ea, ec, eb, ed, wh_mean = pl.pallas_call(
        functools.partial(_proj_body, hid=hid),
        grid=(heads,),
        in_specs=[
            pl.BlockSpec((n, fin), lambda h: (0, 0)),
            pl.BlockSpec((1, fin, hid), lambda h: (h, 0, 0)),
            pl.BlockSpec((heads, hid), lambda h: (0, 0)),
            pl.BlockSpec((heads, hid), lambda h: (0, 0)),
        ],
        out_specs=[
            pl.BlockSpec((1, n, 2 * hid), lambda h: (h, 0, 0)),
            pl.BlockSpec((1, n, 1), lambda h: (h, 0, 0)),
            pl.BlockSpec((1, n, 1), lambda h: (h, 0, 0)),
            pl.BlockSpec((1, n, 1), lambda h: (h, 0, 0)),
            pl.BlockSpec((1, n, 1), lambda h: (h, 0, 0)),
            pl.BlockSpec((1, 1, hid), lambda h: (h, 0, 0)),
        ],
        out_shape=[
            jax.ShapeDtypeStruct((heads, n, 2 * hid), jnp.bfloat16),
            jax.ShapeDtypeStruct((heads, n, 1), jnp.bfloat16),
            jax.ShapeDtypeStruct((heads, n, 1), jnp.bfloat16),
            jax.ShapeDtypeStruct((heads, n, 1), jnp.bfloat16),
            jax.ShapeDtypeStruct((heads, n, 1), jnp.bfloat16),
            jax.ShapeDtypeStruct((heads, 1, hid), jnp.float32),
        ],
    )(x, W, a_src, a_dst)
    eb = eb.reshape(heads, 1, n)
    ed = ed.reshape(heads, 1, n)

    out = pl.pallas_call(
        functools.partial(_attn_body, bm=bm, heads=heads, hid=hid),
        grid=(n // bm,),
        in_specs=[
            pl.BlockSpec((bm, n), lambda i: (i, 0)),
            pl.BlockSpec((heads, n, 2 * hid), lambda i: (0, 0, 0)),
            pl.BlockSpec((heads, n, 1), lambda i: (0, 0, 0)),
            pl.BlockSpec((heads, n, 1), lambda i: (0, 0, 0)),
            pl.BlockSpec((heads, 1, n), lambda i: (0, 0, 0)),
            pl.BlockSpec((heads, 1, n), lambda i: (0, 0, 0)),
            pl.BlockSpec((heads, 1, hid), lambda i: (0, 0, 0)),
        ],
        out_specs=pl.BlockSpec((bm, heads * hid), lambda i: (i, 0)),
        out_shape=jax.ShapeDtypeStruct((n, heads * hid), jnp.float32),
    )(adj_bf, whe, ea, ec, eb, ed, wh_mean)
    return out


@jax.jit
def kernel(features, adj, W1, a_src1, a_dst1, W2, a_src2, a_dst2, W3, a_src3, a_dst3):
    adj_bf = adj.astype(jnp.bfloat16)
    x = _gat_layer(features, adj_bf, W1, a_src1, a_dst1)
    x = _gat_layer(x, adj_bf, W2, a_src2, a_dst2)
    x = _gat_layer(x, adj_bf, W3, a_src3, a_dst3)
    return x


# speedup vs baseline: 2.9175x; 1.0482x over previous
import functools

import jax
import jax.numpy as jnp
from jax.experimental import pallas as pl

_BM = 512


def _proj_body(x_ref, w_ref, asrc_ref, adst_ref,
               whe_ref, ea_ref, ec_ref, eb_ref, ed_ref, mean_ref, *, hid):
    n = x_ref.shape[0]
    h = pl.program_id(0)
    wh = jnp.dot(x_ref[...], w_ref[0], preferred_element_type=jnp.float32)
    whe_ref[0, :, :hid] = wh.astype(jnp.bfloat16)
    col = jax.lax.broadcasted_iota(jnp.int32, (n, hid), 1)
    whe_ref[0, :, hid:] = jnp.where(col == 0, 1.0, 0.0).astype(jnp.bfloat16)
    asrc = asrc_ref[h][:, None]
    adst = adst_ref[h][:, None]
    aboth = jnp.concatenate([asrc, adst], axis=1)
    fs = jnp.dot(wh, aboth, preferred_element_type=jnp.float32)
    f_src = fs[:, 0:1]
    f_dst = fs[:, 1:2]
    mh = jnp.max(f_dst)
    t = f_src + mh
    c = jnp.maximum(t, 0.2 * t)
    ea_ref[0] = jnp.exp(f_src - c).astype(jnp.bfloat16)
    ec_ref[0] = jnp.exp(0.2 * f_src - c).astype(jnp.bfloat16)
    eb_ref[0] = jnp.exp(f_dst).astype(jnp.bfloat16)
    ed_ref[0] = jnp.exp(0.2 * f_dst).astype(jnp.bfloat16)
    mean_ref[0] = jnp.mean(wh, axis=0, keepdims=True)


def _attn_body(adj_ref, whe_ref, ea_ref, ec_ref, eb_ref, ed_ref, mean_ref,
               out_ref, *, bm, heads, hid):
    i = pl.program_id(0)
    adj = adj_ref[...]
    for h in range(heads):
        ea = ea_ref[h, pl.ds(i * bm, bm), :]
        ec = ec_ref[h, pl.ds(i * bm, bm), :]
        eb = eb_ref[h]
        ed = ed_ref[h]
        q = jnp.maximum(ea * eb, ec * ed)
        p = adj * q
        pv = jnp.dot(p, whe_ref[h], preferred_element_type=jnp.float32)
        denom = pv[:, hid : hid + 1]
        o = jnp.where(denom > 0.0, pv[:, :hid] / denom, mean_ref[h])
        o = jnp.where(o > 0.0, o, jnp.exp(o) - 1.0)
        out_ref[:, h * hid : (h + 1) * hid] = o


def _gat_layer(x, adj_bf, W, a_src, a_dst):
    n = x.shape[0]
    fin = x.shape[1]
    heads, _, hid = W.shape
    bm = min(_BM, n)

    whe, ea, ec, eb, ed, wh_mean = pl.pallas_call(
        functools.partial(_proj_body, hid=hid),
        grid=(heads,),
        in_specs=[
            pl.BlockSpec((n, fin), lambda h: (0, 0)),
            pl.BlockSpec((1, fin, hid), lambda h: (h, 0, 0)),
            pl.BlockSpec((heads, hid), lambda h: (0, 0)),
            pl.BlockSpec((heads, hid), lambda h: (0, 0)),
        ],
        out_specs=[
            pl.BlockSpec((1, n, 2 * hid), lambda h: (h, 0, 0)),
            pl.BlockSpec((1, n, 1), lambda h: (h, 0, 0)),
            pl.BlockSpec((1, n, 1), lambda h: (h, 0, 0)),
            pl.BlockSpec((1, n, 1), lambda h: (h, 0, 0)),
            pl.BlockSpec((1, n, 1), lambda h: (h, 0, 0)),
            pl.BlockSpec((1, 1, hid), lambda h: (h, 0, 0)),
        ],
        out_shape=[
            jax.ShapeDtypeStruct((heads, n, 2 * hid), jnp.bfloat16),
            jax.ShapeDtypeStruct((heads, n, 1), jnp.bfloat16),
            jax.ShapeDtypeStruct((heads, n, 1), jnp.bfloat16),
            jax.ShapeDtypeStruct((heads, n, 1), jnp.bfloat16),
            jax.ShapeDtypeStruct((heads, n, 1), jnp.bfloat16),
            jax.ShapeDtypeStruct((heads, 1, hid), jnp.float32),
        ],
    )(x, W, a_src, a_dst)
    eb = eb.reshape(heads, 1, n)
    ed = ed.reshape(heads, 1, n)

    out = pl.pallas_call(
        functools.partial(_attn_body, bm=bm, heads=heads, hid=hid),
        grid=(n // bm,),
        in_specs=[
            pl.BlockSpec((bm, n), lambda i: (i, 0)),
            pl.BlockSpec((heads, n, 2 * hid), lambda i: (0, 0, 0)),
            pl.BlockSpec((heads, n, 1), lambda i: (0, 0, 0)),
            pl.BlockSpec((heads, n, 1), lambda i: (0, 0, 0)),
            pl.BlockSpec((heads, 1, n), lambda i: (0, 0, 0)),
            pl.BlockSpec((heads, 1, n), lambda i: (0, 0, 0)),
            pl.BlockSpec((heads, 1, hid), lambda i: (0, 0, 0)),
        ],
        out_specs=pl.BlockSpec((bm, heads * hid), lambda i: (i, 0)),
        out_shape=jax.ShapeDtypeStruct((n, heads * hid), jnp.float32),
    )(adj_bf, whe, ea, ec, eb, ed, wh_mean)
    return out


@jax.jit
def kernel(features, adj, W1, a_src1, a_dst1, W2, a_src2, a_dst2, W3, a_src3, a_dst3):
    adj_bf = adj.astype(jnp.bfloat16)
    x = _gat_layer(features, adj_bf, W1, a_src1, a_dst1)
    x = _gat_layer(x, adj_bf, W2, a_src2, a_dst2)
    x = _gat_layer(x, adj_bf, W3, a_src3, a_dst3)
    return x


# speedup vs baseline: 3.0459x; 1.0440x over previous
import functools

import jax
import jax.numpy as jnp
from jax.experimental import pallas as pl

_BM = 512


def _proj_body(x_ref, w_ref, asrc_ref, adst_ref,
               whe_ref, g_ref, eb_ref, ed_ref, mean_ref, *, hid):
    n = x_ref.shape[0]
    h = pl.program_id(0)
    wh = jnp.dot(x_ref[...], w_ref[0], preferred_element_type=jnp.float32)
    whe_ref[0, :, :hid] = wh.astype(jnp.bfloat16)
    col = jax.lax.broadcasted_iota(jnp.int32, (n, hid), 1)
    whe_ref[0, :, hid:] = jnp.where(col == 0, 1.0, 0.0).astype(jnp.bfloat16)
    asrc = asrc_ref[h][:, None]
    adst = adst_ref[h][:, None]
    aboth = jnp.concatenate([asrc, adst], axis=1)
    fs = jnp.dot(wh, aboth, preferred_element_type=jnp.float32)
    f_src = fs[:, 0:1]
    f_dst = fs[:, 1:2]
    g_ref[0] = jnp.exp(-0.8 * f_src)
    eb_ref[0] = jnp.exp(f_dst)
    ed_ref[0] = jnp.exp(0.2 * f_dst)
    mean_ref[0] = jnp.mean(wh, axis=0, keepdims=True)


def _attn_body(adj_ref, whe_ref, g_ref, eb_ref, ed_ref, mean_ref,
               out_ref, *, bm, heads, hid):
    i = pl.program_id(0)
    adj = adj_ref[...]
    for h in range(heads):
        g = g_ref[h, pl.ds(i * bm, bm), :]
        eb = eb_ref[h]
        ed = ed_ref[h]
        q = jnp.maximum(eb, g * ed)
        p = adj * q
        pv = jnp.dot(p, whe_ref[h], preferred_element_type=jnp.float32)
        denom = pv[:, hid : hid + 1]
        o = jnp.where(denom > 0.0, pv[:, :hid] / denom, mean_ref[h])
        o = jnp.where(o > 0.0, o, jnp.exp(o) - 1.0)
        out_ref[:, h * hid : (h + 1) * hid] = o


def _gat_layer(x, adj_bf, W, a_src, a_dst):
    n = x.shape[0]
    fin = x.shape[1]
    heads, _, hid = W.shape
    bm = min(_BM, n)

    whe, g, eb, ed, wh_mean = pl.pallas_call(
        functools.partial(_proj_body, hid=hid),
        grid=(heads,),
        in_specs=[
            pl.BlockSpec((n, fin), lambda h: (0, 0)),
            pl.BlockSpec((1, fin, hid), lambda h: (h, 0, 0)),
            pl.BlockSpec((heads, hid), lambda h: (0, 0)),
            pl.BlockSpec((heads, hid), lambda h: (0, 0)),
        ],
        out_specs=[
            pl.BlockSpec((1, n, 2 * hid), lambda h: (h, 0, 0)),
            pl.BlockSpec((1, n, 1), lambda h: (h, 0, 0)),
            pl.BlockSpec((1, n, 1), lambda h: (h, 0, 0)),
            pl.BlockSpec((1, n, 1), lambda h: (h, 0, 0)),
            pl.BlockSpec((1, 1, hid), lambda h: (h, 0, 0)),
        ],
        out_shape=[
            jax.ShapeDtypeStruct((heads, n, 2 * hid), jnp.bfloat16),
            jax.ShapeDtypeStruct((heads, n, 1), jnp.float32),
            jax.ShapeDtypeStruct((heads, n, 1), jnp.float32),
            jax.ShapeDtypeStruct((heads, n, 1), jnp.float32),
            jax.ShapeDtypeStruct((heads, 1, hid), jnp.float32),
        ],
    )(x, W, a_src, a_dst)
    g = g.astype(jnp.bfloat16)
    eb = eb.reshape(heads, 1, n).astype(jnp.bfloat16)
    ed = ed.reshape(heads, 1, n).astype(jnp.bfloat16)

    out = pl.pallas_call(
        functools.partial(_attn_body, bm=bm, heads=heads, hid=hid),
        grid=(n // bm,),
        in_specs=[
            pl.BlockSpec((bm, n), lambda i: (i, 0)),
            pl.BlockSpec((heads, n, 2 * hid), lambda i: (0, 0, 0)),
            pl.BlockSpec((heads, n, 1), lambda i: (0, 0, 0)),
            pl.BlockSpec((heads, 1, n), lambda i: (0, 0, 0)),
            pl.BlockSpec((heads, 1, n), lambda i: (0, 0, 0)),
            pl.BlockSpec((heads, 1, hid), lambda i: (0, 0, 0)),
        ],
        out_specs=pl.BlockSpec((bm, heads * hid), lambda i: (i, 0)),
        out_shape=jax.ShapeDtypeStruct((n, heads * hid), jnp.float32),
    )(adj_bf, whe, g, eb, ed, wh_mean)
    return out


@jax.jit
def kernel(features, adj, W1, a_src1, a_dst1, W2, a_src2, a_dst2, W3, a_src3, a_dst3):
    adj_bf = adj.astype(jnp.bfloat16)
    x = _gat_layer(features, adj_bf, W1, a_src1, a_dst1)
    x = _gat_layer(x, adj_bf, W2, a_src2, a_dst2)
    x = _gat_layer(x, adj_bf, W3, a_src3, a_dst3)
    return x
